# baseline (speedup 1.0000x reference)
import math

import jax
import jax.numpy as jnp
from jax import lax
from jax.experimental import pallas as pl
from jax.experimental.pallas import tpu as pltpu
from jax.experimental.pallas import tpu_sc as plsc

VOCAB = 1000000
DMODEL = 64
SCALE = math.sqrt(DMODEL)

_INFO = plsc.get_sparse_core_info()
NC = _INFO.num_cores
NS = _INFO.num_subcores
NW = NC * NS
LANES = 16

CHUNK = 128
NBUF = 4
DH = DMODEL // 8
NJ = 200


def _transpose_scale(rows_b, trans_b):
    riota = lax.iota(jnp.int32, 16)

    @plsc.parallel_loop(0, DMODEL, step=1, unroll=4)
    def _(d):
        dh = d // 8
        dl = d % 8
        col = jnp.full((16,), d, jnp.int32)
        for ilg in range(CHUNK // LANES):
            vals = plsc.load_gather(rows_b, [riota + ilg * LANES, col])
            trans_b[dh, dl, pl.ds(ilg * LANES, LANES)] = vals * SCALE


def _make_kernel():
    mesh = plsc.VectorSubcoreMesh(core_axis_name="c", subcore_axis_name="s")

    def body(xt_hbm, table_hbm, out_hbm, idx_all, rows_v, trans_v, gat_sems,
             st_sems):
        wid = lax.axis_index("s") * NC + lax.axis_index("c")

        pltpu.sync_copy(xt_hbm.at[:, wid], idx_all)

        def gat_start(j, slot):
            pltpu.async_copy(table_hbm.at[idx_all.at[j]], rows_v.at[slot],
                             gat_sems.at[slot])

        def gat_wait(j, slot):
            pltpu.make_async_copy(table_hbm.at[idx_all.at[j]],
                                  rows_v.at[slot], gat_sems.at[slot]).wait()

        def st_start(j, slot):
            pltpu.async_copy(trans_v.at[slot], out_hbm.at[j, :, wid],
                             st_sems.at[slot])

        def st_wait(j, slot):
            pltpu.make_async_copy(trans_v.at[slot], out_hbm.at[j, :, wid],
                                  st_sems.at[slot]).wait()

        def ts(slot):
            _transpose_scale(rows_v.at[slot], trans_v.at[slot])

        gat_start(0, 0)
        for j in range(1, NBUF):
            gat_start(j, j)
            gat_wait(j - 1, j - 1)
            ts(j - 1)
            st_start(j - 1, j - 1)

        def outer(jq, carry):
            for b in range(NBUF):
                j = jq * NBUF + b
                st_wait(j - NBUF, b)
                gat_start(j, b)
                pb = (b - 1) % NBUF
                gat_wait(j - 1, pb)
                ts(pb)
                st_start(j - 1, pb)
            return carry

        lax.fori_loop(1, NJ // NBUF, outer, 0)

        last = NBUF - 1
        gat_wait(NJ - 1, last)
        ts(last)
        st_start(NJ - 1, last)
        for b in range(NBUF):
            st_wait(NJ - NBUF + b, b)

    return pl.kernel(
        body,
        out_type=jax.ShapeDtypeStruct((NJ, DH, NW, 8, CHUNK), jnp.float32),
        mesh=mesh,
        scratch_types=[
            pltpu.VMEM((NJ, CHUNK), jnp.int32),
            pltpu.VMEM((NBUF, CHUNK, DMODEL), jnp.float32),
            pltpu.VMEM((NBUF, DH, 8, CHUNK), jnp.float32),
            pltpu.SemaphoreType.DMA((NBUF,)),
            pltpu.SemaphoreType.DMA((NBUF,)),
        ],
        compiler_params=pltpu.CompilerParams(use_tc_tiling_on_sc=False,
                                             needs_layout_passes=False),
    )


def kernel(x, table):
    n_i, n_j = x.shape
    assert n_i == NW * CHUNK and n_j == NJ
    xt = x.astype(jnp.int32).transpose((1, 0)).reshape(NJ, NW, CHUNK)
    out5 = _make_kernel()(xt, table)
    return out5.transpose((2, 4, 0, 1, 3)).reshape(n_i, n_j, DMODEL)


# speedup vs baseline: 1.5728x; 1.5728x over previous
import math

import jax
import jax.numpy as jnp
from jax import lax
from jax.experimental import pallas as pl
from jax.experimental.pallas import tpu as pltpu
from jax.experimental.pallas import tpu_sc as plsc

VOCAB = 1000000
DMODEL = 64
SCALE = math.sqrt(DMODEL)

_INFO = plsc.get_sparse_core_info()
NC = _INFO.num_cores
NS = _INFO.num_subcores
NW = NC * NS
LANES = 16
NV = 8

CHUNK = 128
NBUF = 4
DH = DMODEL // 8
NJ = 200


PITCH = 129


def _transpose_scale(rows_b, cm_b, trans_b):
    piota = lax.iota(jnp.int32, 16) * PITCH

    @plsc.parallel_loop(0, CHUNK, step=1, unroll=4)
    def _(r):
        rbase = piota + r
        for k in range(DMODEL // LANES):
            vals = rows_b[r, pl.ds(k * LANES, LANES)]
            plsc.store_scatter(cm_b, [rbase + k * LANES * PITCH],
                               vals * SCALE)

    @plsc.parallel_loop(0, DMODEL, step=1, unroll=4)
    def _(d):
        dh = d // 8
        dl = d % 8
        base = d * PITCH
        for g in range(NV):
            trans_b[dh, dl, pl.ds(g * LANES, LANES)] = (
                cm_b[pl.ds(base + g * LANES, LANES)])


def _make_kernel():
    mesh = plsc.VectorSubcoreMesh(core_axis_name="c", subcore_axis_name="s")

    def body(xt_hbm, table_hbm, out_hbm, idx_all, rows_v, cm_v, trans_v,
             gat_sems, st_sems):
        wid = lax.axis_index("s") * NC + lax.axis_index("c")

        pltpu.sync_copy(xt_hbm.at[:, wid], idx_all)

        def gat_start(j, slot):
            pltpu.async_copy(table_hbm.at[idx_all.at[j]], rows_v.at[slot],
                             gat_sems.at[slot])

        def gat_wait(j, slot):
            pltpu.make_async_copy(table_hbm.at[idx_all.at[j]],
                                  rows_v.at[slot], gat_sems.at[slot]).wait()

        def st_start(j, slot):
            pltpu.async_copy(trans_v.at[slot], out_hbm.at[j, :, wid],
                             st_sems.at[slot])

        def st_wait(j, slot):
            pltpu.make_async_copy(trans_v.at[slot], out_hbm.at[j, :, wid],
                                  st_sems.at[slot]).wait()

        def ts(slot):
            _transpose_scale(rows_v.at[slot], cm_v.at[slot],
                             trans_v.at[slot])

        gat_start(0, 0)
        for j in range(1, NBUF):
            gat_start(j, j)
            gat_wait(j - 1, j - 1)
            ts(j - 1)
            st_start(j - 1, j - 1)

        def outer(jq, carry):
            for b in range(NBUF):
                j = jq * NBUF + b
                st_wait(j - NBUF, b)
                gat_start(j, b)
                pb = (b - 1) % NBUF
                gat_wait(j - 1, pb)
                ts(pb)
                st_start(j - 1, pb)
            return carry

        lax.fori_loop(1, NJ // NBUF, outer, 0)

        last = NBUF - 1
        gat_wait(NJ - 1, last)
        ts(last)
        st_start(NJ - 1, last)
        for b in range(NBUF):
            st_wait(NJ - NBUF + b, b)

    return pl.kernel(
        body,
        out_type=jax.ShapeDtypeStruct((NJ, DH, NW, 8, CHUNK), jnp.float32),
        mesh=mesh,
        scratch_types=[
            pltpu.VMEM((NJ, CHUNK), jnp.int32),
            pltpu.VMEM((NBUF, CHUNK, DMODEL), jnp.float32),
            pltpu.VMEM((NBUF, DMODEL * PITCH), jnp.float32),
            pltpu.VMEM((NBUF, DH, 8, CHUNK), jnp.float32),
            pltpu.SemaphoreType.DMA((NBUF,)),
            pltpu.SemaphoreType.DMA((NBUF,)),
        ],
        compiler_params=pltpu.CompilerParams(use_tc_tiling_on_sc=False,
                                             needs_layout_passes=False),
    )


def kernel(x, table):
    n_i, n_j = x.shape
    assert n_i == NW * CHUNK and n_j == NJ
    xt = x.astype(jnp.int32).transpose((1, 0)).reshape(NJ, NW, CHUNK)
    out5 = _make_kernel()(xt, table)
    return out5.transpose((2, 4, 0, 1, 3)).reshape(n_i, n_j, DMODEL)


# speedup vs baseline: 2.4724x; 1.5720x over previous
import math

import jax
import jax.numpy as jnp
from jax import lax
from jax.experimental import pallas as pl
from jax.experimental.pallas import tpu as pltpu
from jax.experimental.pallas import tpu_sc as plsc

VOCAB = 1000000
DMODEL = 64
SCALE = math.sqrt(DMODEL)

_INFO = plsc.get_sparse_core_info()
NC = _INFO.num_cores
NS = _INFO.num_subcores
NW = NC * NS
LANES = 16
NV = 8

CHUNK = 128
NBUF = 4
DH = DMODEL // 8
NJ = 200


PITCH = 129


def _transpose_scale(rows_b, cm_b, trans_b):
    piota = lax.iota(jnp.int32, 16) * PITCH

    @plsc.parallel_loop(0, CHUNK, step=1, unroll=4)
    def _(r):
        rbase = piota + r
        for k in range(DMODEL // LANES):
            vals = rows_b[r, pl.ds(k * LANES, LANES)]
            plsc.store_scatter(cm_b, [rbase + k * LANES * PITCH],
                               vals * SCALE)

    @plsc.parallel_loop(0, DMODEL, step=1, unroll=4)
    def _(d):
        dh = d // 8
        dl = d % 8
        base = d * PITCH
        for g in range(NV):
            trans_b[dh, dl, pl.ds(g * LANES, LANES)] = (
                cm_b[pl.ds(base + g * LANES, LANES)])


NT_FULL = VOCAB // CHUNK
TAIL = VOCAB - NT_FULL * CHUNK


def _make_format_kernel():
    mesh = plsc.VectorSubcoreMesh(core_axis_name="c", subcore_axis_name="s")

    def body(tt_hbm, tail_hbm, out_hbm, blk0, blk1, cm0, cm1, ob0, ob1,
             tail_v, in_sems, out_sems):
        wid = lax.axis_index("s") * NC + lax.axis_index("c")
        base = 244 * wid + jnp.minimum(wid, 4)
        blks, cms, obs = (blk0, blk1), (cm0, cm1), (ob0, ob1)

        def in_start(t, slot):
            pltpu.async_copy(tt_hbm.at[:, pl.ds(t * CHUNK, CHUNK)],
                             blks[slot], in_sems.at[slot])

        def in_wait(t, slot):
            pltpu.make_async_copy(tt_hbm.at[:, pl.ds(t * CHUNK, CHUNK)],
                                  blks[slot], in_sems.at[slot]).wait()

        def st_start(t, slot):
            pltpu.async_copy(obs[slot],
                             out_hbm.at[pl.ds(t * CHUNK * DMODEL,
                                              CHUNK * DMODEL)],
                             out_sems.at[slot])

        def st_wait(t, slot):
            pltpu.make_async_copy(obs[slot],
                                  out_hbm.at[pl.ds(t * CHUNK * DMODEL,
                                                   CHUNK * DMODEL)],
                                  out_sems.at[slot]).wait()

        piota = lax.iota(jnp.int32, 16) * PITCH
        pvs = [piota + g * LANES * PITCH for g in range(CHUNK // LANES)]

        def compute(slot):
            blk_b = blks[slot]
            cm_b = cms[slot]
            ob_b = obs[slot]

            @plsc.parallel_loop(0, DMODEL, step=1, unroll=4)
            def _(d):
                for g in range(CHUNK // LANES):
                    vals = blk_b[d, pl.ds(g * LANES, LANES)]
                    plsc.store_scatter(cm_b, [pvs[g] + d], vals)

            @plsc.parallel_loop(0, CHUNK, step=1, unroll=4)
            def _(u):
                cb = u * PITCH
                obb = u * DMODEL
                for k in range(DMODEL // LANES):
                    ob_b[pl.ds(obb + k * LANES, LANES)] = (
                        cm_b[pl.ds(cb + k * LANES, LANES)])

        def step(i, slot, with_st_wait, with_prefetch):
            t = base + i
            in_wait(t, slot)
            if with_prefetch:
                in_start(jnp.minimum(t + 1, NT_FULL - 1), 1 - slot)
            if with_st_wait:
                st_wait(t - 2, slot)
            compute(slot)
            st_start(t, slot)

        in_start(base, 0)
        step(0, 0, False, True)
        step(1, 1, False, True)

        def pair(g, carry):
            step(2 * g, 0, True, True)
            step(2 * g + 1, 1, True, True)
            return carry

        lax.fori_loop(1, 122, pair, 0)

        @pl.when(wid < 4)
        def _():
            step(244, 0, True, False)

        @pl.when(wid >= 4)
        def _():
            in_wait(base, 0)

        st_wait(base, 0)
        st_wait(base, 1)

        @pl.when(wid == NW - 1)
        def _():
            pltpu.sync_copy(tail_hbm, tail_v)
            pltpu.sync_copy(tail_v,
                            out_hbm.at[pl.ds(NT_FULL * CHUNK * DMODEL,
                                             TAIL * DMODEL)])

    return pl.kernel(
        body,
        out_type=jax.ShapeDtypeStruct((VOCAB * DMODEL,), jnp.float32),
        mesh=mesh,
        scratch_types=[
            pltpu.VMEM((DMODEL, CHUNK), jnp.float32),
            pltpu.VMEM((DMODEL, CHUNK), jnp.float32),
            pltpu.VMEM((CHUNK * PITCH,), jnp.float32),
            pltpu.VMEM((CHUNK * PITCH,), jnp.float32),
            pltpu.VMEM((CHUNK * DMODEL,), jnp.float32),
            pltpu.VMEM((CHUNK * DMODEL,), jnp.float32),
            pltpu.VMEM((TAIL * DMODEL,), jnp.float32),
            pltpu.SemaphoreType.DMA((2,)),
            pltpu.SemaphoreType.DMA((2,)),
        ],
        compiler_params=pltpu.CompilerParams(use_tc_tiling_on_sc=True,
                                             needs_layout_passes=False),
    )


def _make_kernel():
    mesh = plsc.VectorSubcoreMesh(core_axis_name="c", subcore_axis_name="s")

    def body(xt_hbm, table_hbm, out_hbm, idx_all, rows_v, cm_v, trans_v,
             gat_sems, st_sems):
        wid = lax.axis_index("s") * NC + lax.axis_index("c")

        pltpu.sync_copy(xt_hbm.at[:, wid], idx_all)

        def gat_start(j, slot):
            pltpu.async_copy(table_hbm.at[idx_all.at[j]], rows_v.at[slot],
                             gat_sems.at[slot])

        def gat_wait(j, slot):
            pltpu.make_async_copy(table_hbm.at[idx_all.at[j]],
                                  rows_v.at[slot], gat_sems.at[slot]).wait()

        def st_start(j, slot):
            pltpu.async_copy(trans_v.at[slot], out_hbm.at[j, :, wid],
                             st_sems.at[slot])

        def st_wait(j, slot):
            pltpu.make_async_copy(trans_v.at[slot], out_hbm.at[j, :, wid],
                                  st_sems.at[slot]).wait()

        def ts(slot):
            _transpose_scale(rows_v.at[slot], cm_v.at[slot],
                             trans_v.at[slot])

        gat_start(0, 0)
        for j in range(1, NBUF):
            gat_start(j, j)
            gat_wait(j - 1, j - 1)
            ts(j - 1)
            st_start(j - 1, j - 1)

        def outer(jq, carry):
            for b in range(NBUF):
                j = jq * NBUF + b
                st_wait(j - NBUF, b)
                gat_start(j, b)
                pb = (b - 1) % NBUF
                gat_wait(j - 1, pb)
                ts(pb)
                st_start(j - 1, pb)
            return carry

        lax.fori_loop(1, NJ // NBUF, outer, 0)

        last = NBUF - 1
        gat_wait(NJ - 1, last)
        ts(last)
        st_start(NJ - 1, last)
        for b in range(NBUF):
            st_wait(NJ - NBUF + b, b)

    return pl.kernel(
        body,
        out_type=jax.ShapeDtypeStruct((NJ, DH, NW, 8, CHUNK), jnp.float32),
        mesh=mesh,
        scratch_types=[
            pltpu.VMEM((NJ, CHUNK), jnp.int32),
            pltpu.VMEM((NBUF, CHUNK, DMODEL), jnp.float32),
            pltpu.VMEM((NBUF, DMODEL * PITCH), jnp.float32),
            pltpu.VMEM((NBUF, DH, 8, CHUNK), jnp.float32),
            pltpu.SemaphoreType.DMA((NBUF,)),
            pltpu.SemaphoreType.DMA((NBUF,)),
        ],
        compiler_params=pltpu.CompilerParams(use_tc_tiling_on_sc=False,
                                             needs_layout_passes=False),
    )


def kernel(x, table):
    n_i, n_j = x.shape
    assert n_i == NW * CHUNK and n_j == NJ
    xt = x.astype(jnp.int32).transpose((1, 0)).reshape(NJ, NW, CHUNK)
    tt = table.transpose((1, 0))
    tail = table[NT_FULL * CHUNK:].reshape(TAIL * DMODEL)
    tflat = _make_format_kernel()(tt, tail)
    out5 = _make_kernel()(xt, tflat.reshape(VOCAB, DMODEL))
    return out5.transpose((2, 4, 0, 1, 3)).reshape(n_i, n_j, DMODEL)


# speedup vs baseline: 2.4778x; 1.0022x over previous
import math

import jax
import jax.numpy as jnp
from jax import lax
from jax.experimental import pallas as pl
from jax.experimental.pallas import tpu as pltpu
from jax.experimental.pallas import tpu_sc as plsc

VOCAB = 1000000
DMODEL = 64
SCALE = math.sqrt(DMODEL)

_INFO = plsc.get_sparse_core_info()
NC = _INFO.num_cores
NS = _INFO.num_subcores
NW = NC * NS
LANES = 16
NV = 8

CHUNK = 128
NBUF = 4
DH = DMODEL // 8
NJ = 200


PITCH = 129


def _transpose_scale(rows_b, cm_b, trans_b):
    piota = lax.iota(jnp.int32, 16) * PITCH

    @plsc.parallel_loop(0, CHUNK, step=1, unroll=8)
    def _(r):
        rbase = piota + r
        for k in range(DMODEL // LANES):
            vals = rows_b[r, pl.ds(k * LANES, LANES)]
            plsc.store_scatter(cm_b, [rbase + k * LANES * PITCH],
                               vals * SCALE)

    @plsc.parallel_loop(0, DMODEL, step=1, unroll=8)
    def _(d):
        dh = d // 8
        dl = d % 8
        base = d * PITCH
        for g in range(NV):
            trans_b[dh, dl, pl.ds(g * LANES, LANES)] = (
                cm_b[pl.ds(base + g * LANES, LANES)])


NT_FULL = VOCAB // CHUNK
TAIL = VOCAB - NT_FULL * CHUNK


def _make_format_kernel():
    mesh = plsc.VectorSubcoreMesh(core_axis_name="c", subcore_axis_name="s")

    def body(tt_hbm, tail_hbm, out_hbm, blk0, blk1, cm0, cm1, ob0, ob1,
             tail_v, in_sems, out_sems):
        wid = lax.axis_index("s") * NC + lax.axis_index("c")
        base = 244 * wid + jnp.minimum(wid, 4)
        blks, cms, obs = (blk0, blk1), (cm0, cm1), (ob0, ob1)

        def in_start(t, slot):
            pltpu.async_copy(tt_hbm.at[:, pl.ds(t * CHUNK, CHUNK)],
                             blks[slot], in_sems.at[slot])

        def in_wait(t, slot):
            pltpu.make_async_copy(tt_hbm.at[:, pl.ds(t * CHUNK, CHUNK)],
                                  blks[slot], in_sems.at[slot]).wait()

        def st_start(t, slot):
            pltpu.async_copy(obs[slot],
                             out_hbm.at[pl.ds(t * CHUNK * DMODEL,
                                              CHUNK * DMODEL)],
                             out_sems.at[slot])

        def st_wait(t, slot):
            pltpu.make_async_copy(obs[slot],
                                  out_hbm.at[pl.ds(t * CHUNK * DMODEL,
                                                   CHUNK * DMODEL)],
                                  out_sems.at[slot]).wait()

        piota = lax.iota(jnp.int32, 16) * PITCH
        pvs = [piota + g * LANES * PITCH for g in range(CHUNK // LANES)]

        def compute(slot):
            blk_b = blks[slot]
            cm_b = cms[slot]
            ob_b = obs[slot]

            @plsc.parallel_loop(0, DMODEL, step=1, unroll=8)
            def _(d):
                for g in range(CHUNK // LANES):
                    vals = blk_b[d, pl.ds(g * LANES, LANES)]
                    plsc.store_scatter(cm_b, [pvs[g] + d], vals)

            @plsc.parallel_loop(0, CHUNK, step=1, unroll=8)
            def _(u):
                cb = u * PITCH
                obb = u * DMODEL
                for k in range(DMODEL // LANES):
                    ob_b[pl.ds(obb + k * LANES, LANES)] = (
                        cm_b[pl.ds(cb + k * LANES, LANES)])

        def step(i, slot, with_st_wait, with_prefetch):
            t = base + i
            in_wait(t, slot)
            if with_prefetch:
                in_start(jnp.minimum(t + 1, NT_FULL - 1), 1 - slot)
            if with_st_wait:
                st_wait(t - 2, slot)
            compute(slot)
            st_start(t, slot)

        in_start(base, 0)
        step(0, 0, False, True)
        step(1, 1, False, True)

        def pair(g, carry):
            step(2 * g, 0, True, True)
            step(2 * g + 1, 1, True, True)
            return carry

        lax.fori_loop(1, 122, pair, 0)

        @pl.when(wid < 4)
        def _():
            step(244, 0, True, False)

        @pl.when(wid >= 4)
        def _():
            in_wait(base, 0)

        st_wait(base, 0)
        st_wait(base, 1)

        @pl.when(wid == NW - 1)
        def _():
            pltpu.sync_copy(tail_hbm, tail_v)
            pltpu.sync_copy(tail_v,
                            out_hbm.at[pl.ds(NT_FULL * CHUNK * DMODEL,
                                             TAIL * DMODEL)])

    return pl.kernel(
        body,
        out_type=jax.ShapeDtypeStruct((VOCAB * DMODEL,), jnp.float32),
        mesh=mesh,
        scratch_types=[
            pltpu.VMEM((DMODEL, CHUNK), jnp.float32),
            pltpu.VMEM((DMODEL, CHUNK), jnp.float32),
            pltpu.VMEM((CHUNK * PITCH,), jnp.float32),
            pltpu.VMEM((CHUNK * PITCH,), jnp.float32),
            pltpu.VMEM((CHUNK * DMODEL,), jnp.float32),
            pltpu.VMEM((CHUNK * DMODEL,), jnp.float32),
            pltpu.VMEM((TAIL * DMODEL,), jnp.float32),
            pltpu.SemaphoreType.DMA((2,)),
            pltpu.SemaphoreType.DMA((2,)),
        ],
        compiler_params=pltpu.CompilerParams(use_tc_tiling_on_sc=True,
                                             needs_layout_passes=False),
    )


def _make_kernel():
    mesh = plsc.VectorSubcoreMesh(core_axis_name="c", subcore_axis_name="s")

    def body(xt_hbm, table_hbm, out_hbm, idx_all, rows_v, cm_v, trans_v,
             gat_sems, st_sems):
        wid = lax.axis_index("s") * NC + lax.axis_index("c")

        pltpu.sync_copy(xt_hbm.at[:, wid], idx_all)

        def gat_start(j, slot):
            pltpu.async_copy(table_hbm.at[idx_all.at[j]], rows_v.at[slot],
                             gat_sems.at[slot])

        def gat_wait(j, slot):
            pltpu.make_async_copy(table_hbm.at[idx_all.at[j]],
                                  rows_v.at[slot], gat_sems.at[slot]).wait()

        def st_start(j, slot):
            pltpu.async_copy(trans_v.at[slot], out_hbm.at[j, :, wid],
                             st_sems.at[slot])

        def st_wait(j, slot):
            pltpu.make_async_copy(trans_v.at[slot], out_hbm.at[j, :, wid],
                                  st_sems.at[slot]).wait()

        def ts(slot):
            _transpose_scale(rows_v.at[slot], cm_v.at[slot],
                             trans_v.at[slot])

        gat_start(0, 0)
        for j in range(1, NBUF):
            gat_start(j, j)
            gat_wait(j - 1, j - 1)
            ts(j - 1)
            st_start(j - 1, j - 1)

        def outer(jq, carry):
            for b in range(NBUF):
                j = jq * NBUF + b
                st_wait(j - NBUF, b)
                gat_start(j, b)
                pb = (b - 1) % NBUF
                gat_wait(j - 1, pb)
                ts(pb)
                st_start(j - 1, pb)
            return carry

        lax.fori_loop(1, NJ // NBUF, outer, 0)

        last = NBUF - 1
        gat_wait(NJ - 1, last)
        ts(last)
        st_start(NJ - 1, last)
        for b in range(NBUF):
            st_wait(NJ - NBUF + b, b)

    return pl.kernel(
        body,
        out_type=jax.ShapeDtypeStruct((NJ, DH, NW, 8, CHUNK), jnp.float32),
        mesh=mesh,
        scratch_types=[
            pltpu.VMEM((NJ, CHUNK), jnp.int32),
            pltpu.VMEM((NBUF, CHUNK, DMODEL), jnp.float32),
            pltpu.VMEM((NBUF, DMODEL * PITCH), jnp.float32),
            pltpu.VMEM((NBUF, DH, 8, CHUNK), jnp.float32),
            pltpu.SemaphoreType.DMA((NBUF,)),
            pltpu.SemaphoreType.DMA((NBUF,)),
        ],
        compiler_params=pltpu.CompilerParams(use_tc_tiling_on_sc=False,
                                             needs_layout_passes=False),
    )


def kernel(x, table):
    n_i, n_j = x.shape
    assert n_i == NW * CHUNK and n_j == NJ
    xt = x.astype(jnp.int32).transpose((1, 0)).reshape(NJ, NW, CHUNK)
    tt = table.transpose((1, 0))
    tail = table[NT_FULL * CHUNK:].reshape(TAIL * DMODEL)
    tflat = _make_format_kernel()(tt, tail)
    out5 = _make_kernel()(xt, tflat.reshape(VOCAB, DMODEL))
    return out5.transpose((2, 4, 0, 1, 3)).reshape(n_i, n_j, DMODEL)
